# baseline (speedup 1.0000x reference)
import functools

import jax
import jax.numpy as jnp
from jax import lax
from jax.experimental import pallas as pl
from jax.experimental.pallas import tpu as pltpu
from jax.experimental.pallas import tpu_sc as plsc

N = 10000
E = 320000
D = 128
ED = 16
G = 16
C = 10

NC = 2
NS = 16
CH = 80
CR_TOT = 4096
E_PAD = CR_TOT * CH
NCH0 = 168
NCH1 = 88
R0 = NS * NCH0
IG = 8
NG0 = NCH0 // IG
NG1 = NCH1 // IG
N_PAD = 10112
ZROWS = N_PAD // NS
ZB = 32

RA = 2000
RN = 1000


def _edge_embed_body(a_ref, w_ref, b_ref, e_ref):
    e_ref[...] = (jnp.dot(a_ref[...], w_ref[...], preferred_element_type=jnp.float32)
                  + b_ref[...])


def _edge_embed(attr, wt, b):
    return pl.pallas_call(
        _edge_embed_body,
        grid=(E // RA,),
        in_specs=[
            pl.BlockSpec((RA, ED), lambda i: (i, 0)),
            pl.BlockSpec((ED, D), lambda i: (0, 0)),
            pl.BlockSpec((1, D), lambda i: (0, 0)),
        ],
        out_specs=pl.BlockSpec((RA, D), lambda i: (i, 0)),
        out_shape=jax.ShapeDtypeStruct((E_PAD, D), jnp.float32),
    )(attr, wt, b.reshape(1, D))


def _sc_gather_scatter(table, src2d, dst2d, e):
    mesh = plsc.VectorSubcoreMesh(core_axis_name="c", subcore_axis_name="s")

    @functools.partial(
        pl.kernel,
        out_type=jax.ShapeDtypeStruct((NC, N_PAD, D), jnp.float32),
        mesh=mesh,
        scratch_types=[
            pltpu.VMEM((IG, CH), jnp.int32),
            pltpu.VMEM((IG, CH), jnp.int32),
            pltpu.VMEM((CH, D), jnp.float32),
            pltpu.VMEM((CH, D), jnp.float32),
            pltpu.VMEM((CH, D), jnp.float32),
            pltpu.VMEM((CH, D), jnp.float32),
            pltpu.VMEM_SHARED((N_PAD, D), jnp.float32),
        ] + [pltpu.SemaphoreType.DMA] * 6,
    )
    def k(table_hbm, src_hbm, dst_hbm, e_hbm, out_hbm,
          si, di, ev0, ev1, xg0, xg1, agg_sh,
          sg0, sg1, se0, se1, ss0, ss1):
        c = lax.axis_index("c")
        s = lax.axis_index("s")
        ev = (ev0, ev1)
        xg = (xg0, xg1)
        sg = (sg0, sg1)
        se = (se0, se1)
        ss = (ss0, ss1)

        def _zrow(i, carry):
            for cc in range(D // 16):
                ev0[i, pl.ds(cc * 16, 16)] = jnp.zeros((16,), jnp.float32)
            return carry
        lax.fori_loop(0, CH, _zrow, 0)
        zbase = s * ZROWS
        for t in range(ZROWS // CH):
            pltpu.sync_copy(ev0, agg_sh.at[pl.ds(zbase + t * CH, CH)])
        pltpu.sync_copy(ev0.at[pl.ds(0, ZROWS % CH)],
                        agg_sh.at[pl.ds(zbase + (ZROWS // CH) * CH, ZROWS % CH)])
        plsc.subcore_barrier()

        is0 = c == 0
        ng = jnp.where(is0, NG0, NG1)
        base_row = jnp.where(is0, s * NCH0, R0 + s * NCH1)

        def _compute(b):
            def _row(r, rc):
                for cc in range(D // 16):
                    sl = pl.ds(cc * 16, 16)
                    ev[b][r, sl] = jnp.maximum(ev[b][r, sl] + xg[b][r, sl], 0.0)
                return rc
            lax.fori_loop(0, CH, _row, 0, unroll=4)

        def _group(g, carry):
            r0 = base_row + g * IG
            pltpu.sync_copy(src_hbm.at[pl.ds(r0, IG)], si)
            pltpu.sync_copy(dst_hbm.at[pl.ds(r0, IG)], di)
            for p in range(IG // 2):
                a, b2 = 2 * p, 2 * p + 1
                dga = pltpu.async_copy(table_hbm.at[si.at[a]], xg0, sg0)
                dea = pltpu.async_copy(e_hbm.at[pl.ds((r0 + a) * CH, CH)], ev0, se0)
                dgb = pltpu.async_copy(table_hbm.at[si.at[b2]], xg1, sg1)
                deb = pltpu.async_copy(e_hbm.at[pl.ds((r0 + b2) * CH, CH)], ev1, se1)
                dea.wait()
                dga.wait()
                _compute(0)
                dsa = pltpu.make_async_copy(ev0, agg_sh.at[di.at[a]], ss0)
                dsa.start(add=True)
                deb.wait()
                dgb.wait()
                _compute(1)
                dsb = pltpu.make_async_copy(ev1, agg_sh.at[di.at[b2]], ss1)
                dsb.start(add=True)
                dsa.wait()
                dsb.wait()
            return carry
        lax.fori_loop(0, ng, _group, 0)

        plsc.subcore_barrier()
        pltpu.sync_copy(agg_sh.at[pl.ds(s * ZROWS, ZROWS)],
                        out_hbm.at[c].at[pl.ds(s * ZROWS, ZROWS)])

    return k(table, src2d, dst2d, e)


def _node_mlp_body(x_ref, a0_ref, a1_ref, wa_ref, ba_ref, wb_ref, bb_ref, o_ref):
    h = x_ref[...] + a0_ref[0] + a1_ref[0]
    h = jnp.dot(h, wa_ref[...], preferred_element_type=jnp.float32) + ba_ref[...]
    h = jnp.dot(h, wb_ref[...], preferred_element_type=jnp.float32) + bb_ref[...]
    o_ref[...] = jnp.maximum(h, 0.0)


def _node_mlp(x, agg, wat, ba, wbt, bb):
    grid = (N // RN,)
    return pl.pallas_call(
        _node_mlp_body,
        grid=grid,
        in_specs=[
            pl.BlockSpec((RN, D), lambda i: (i, 0)),
            pl.BlockSpec((1, RN, D), lambda i: (0, i, 0)),
            pl.BlockSpec((1, RN, D), lambda i: (1, i, 0)),
            pl.BlockSpec((D, D), lambda i: (0, 0)),
            pl.BlockSpec((1, D), lambda i: (0, 0)),
            pl.BlockSpec((D, D), lambda i: (0, 0)),
            pl.BlockSpec((1, D), lambda i: (0, 0)),
        ],
        out_specs=pl.BlockSpec((RN, D), lambda i: (i, 0)),
        out_shape=jax.ShapeDtypeStruct((N, D), jnp.float32),
    )(x, agg, agg, wat, ba.reshape(1, D), wbt, bb.reshape(1, D))


def _node_mlp_pool_body(x_ref, a0_ref, a1_ref, wa_ref, ba_ref, wb_ref, bb_ref,
                        batch_ref, sums_ref, cnts_ref):
    i = pl.program_id(0)
    h = x_ref[...] + a0_ref[0] + a1_ref[0]
    h = jnp.dot(h, wa_ref[...], preferred_element_type=jnp.float32) + ba_ref[...]
    h = jnp.dot(h, wb_ref[...], preferred_element_type=jnp.float32) + bb_ref[...]
    h = jnp.maximum(h, 0.0)
    b = batch_ref[0, 0, :]
    onehot = (b[:, None] == lax.broadcasted_iota(jnp.int32, (RN, G), 1)).astype(jnp.float32)
    part = lax.dot_general(onehot, h, (((0,), (0,)), ((), ())),
                           preferred_element_type=jnp.float32)
    cnt = jnp.broadcast_to(jnp.sum(onehot, axis=0)[:, None], (G, D))

    @pl.when(i == 0)
    def _():
        sums_ref[...] = jnp.zeros_like(sums_ref)
        cnts_ref[...] = jnp.zeros_like(cnts_ref)
    sums_ref[...] += part
    cnts_ref[...] += cnt


def _node_mlp_pool(h1, agg, wat, ba, wbt, bb, batch3d):
    grid = (N // RN,)
    return pl.pallas_call(
        _node_mlp_pool_body,
        grid=grid,
        in_specs=[
            pl.BlockSpec((RN, D), lambda i: (i, 0)),
            pl.BlockSpec((1, RN, D), lambda i: (0, i, 0)),
            pl.BlockSpec((1, RN, D), lambda i: (1, i, 0)),
            pl.BlockSpec((D, D), lambda i: (0, 0)),
            pl.BlockSpec((1, D), lambda i: (0, 0)),
            pl.BlockSpec((D, D), lambda i: (0, 0)),
            pl.BlockSpec((1, D), lambda i: (0, 0)),
            pl.BlockSpec((1, 1, RN), lambda i: (i, 0, 0)),
        ],
        out_specs=[
            pl.BlockSpec((G, D), lambda i: (0, 0)),
            pl.BlockSpec((G, D), lambda i: (0, 0)),
        ],
        out_shape=[
            jax.ShapeDtypeStruct((G, D), jnp.float32),
            jax.ShapeDtypeStruct((G, D), jnp.float32),
        ],
    )(h1, agg, agg, wat, ba.reshape(1, D), wbt, bb.reshape(1, D), batch3d)


def _head_body(sums_ref, cnts_ref, w_ref, b_ref, o_ref):
    pooled = sums_ref[...] / jnp.maximum(cnts_ref[...], 1.0)
    logits = lax.dot_general(pooled, w_ref[...], (((1,), (1,)), ((), ())),
                             preferred_element_type=jnp.float32) + b_ref[...]
    mask = lax.broadcasted_iota(jnp.int32, (G, G), 1) < C
    logits = jnp.where(mask, logits, -1e30)
    m = jnp.max(logits, axis=1, keepdims=True)
    ez = jnp.exp(logits - m)
    o_ref[...] = ez / jnp.sum(ez, axis=1, keepdims=True)


def _head(sums, cnts, ffn_Wp, ffn_bp):
    return pl.pallas_call(
        _head_body,
        in_specs=[
            pl.BlockSpec((G, D), lambda: (0, 0)),
            pl.BlockSpec((G, D), lambda: (0, 0)),
            pl.BlockSpec((G, D), lambda: (0, 0)),
            pl.BlockSpec((1, G), lambda: (0, 0)),
        ],
        out_specs=pl.BlockSpec((G, G), lambda: (0, 0)),
        out_shape=jax.ShapeDtypeStruct((G, G), jnp.float32),
    )(sums, cnts, ffn_Wp, ffn_bp)


def kernel(x, edge_index, edge_attr, batch,
           le1_W, le1_b, n1a_W, n1a_b, n1b_W, n1b_b,
           le2_W, le2_b, n2a_W, n2a_b, n2b_W, n2b_b,
           ffn_W, ffn_b):
    pad = E_PAD - E
    src2d = jnp.concatenate([edge_index[0], jnp.zeros((pad,), jnp.int32)]).reshape(CR_TOT, CH)
    dst2d = jnp.concatenate([edge_index[1], jnp.full((pad,), N, jnp.int32)]).reshape(CR_TOT, CH)
    batch3d = batch.reshape(N // RN, 1, RN)

    e1 = _edge_embed(edge_attr, le1_W.T, le1_b)
    agg1 = _sc_gather_scatter(x, src2d, dst2d, e1)
    e2 = _edge_embed(edge_attr, le2_W.T, le2_b)
    h1 = _node_mlp(x, agg1, n1a_W.T, n1a_b, n1b_W.T, n1b_b)

    agg2 = _sc_gather_scatter(h1, src2d, dst2d, e2)
    ffn_Wp = jnp.concatenate([ffn_W, jnp.zeros((G - C, D), jnp.float32)])
    ffn_bp = jnp.concatenate([ffn_b, jnp.zeros((G - C,), jnp.float32)]).reshape(1, G)
    sums, cnts = _node_mlp_pool(h1, agg2, n2a_W.T, n2a_b, n2b_W.T, n2b_b, batch3d)

    out = _head(sums, cnts, ffn_Wp, ffn_bp)
    return out[:, :C]


# speedup vs baseline: 1.1644x; 1.1644x over previous
import functools

import jax
import jax.numpy as jnp
from jax import lax
from jax.experimental import pallas as pl
from jax.experimental.pallas import tpu as pltpu
from jax.experimental.pallas import tpu_sc as plsc

N = 10000
E = 320000
D = 128
ED = 16
G = 16
C = 10

NC = 2
NS = 16
CH = 80
CR_TOT = 4096
E_PAD = CR_TOT * CH
NCH0 = 168
NCH1 = 88
R0 = NS * NCH0
IG = 8
NG0 = NCH0 // IG
NG1 = NCH1 // IG
N_PAD = 10112
ZROWS = N_PAD // NS
ZB = 32

RA = 8000
RN = 1000


def _edge_embed_body(a_ref, w_ref, b_ref, e_ref):
    e_ref[...] = (jnp.dot(a_ref[...], w_ref[...], preferred_element_type=jnp.float32)
                  + b_ref[...])


def _edge_embed(attr, wt, b):
    return pl.pallas_call(
        _edge_embed_body,
        grid=(E // RA,),
        in_specs=[
            pl.BlockSpec((RA, ED), lambda i: (i, 0)),
            pl.BlockSpec((ED, D), lambda i: (0, 0)),
            pl.BlockSpec((1, D), lambda i: (0, 0)),
        ],
        out_specs=pl.BlockSpec((RA, D), lambda i: (i, 0)),
        out_shape=jax.ShapeDtypeStruct((E_PAD, D), jnp.float32),
    )(attr, wt, b.reshape(1, D))


def _sc_gather_scatter(table, src2d, dst2d, e):
    mesh = plsc.VectorSubcoreMesh(core_axis_name="c", subcore_axis_name="s")

    @functools.partial(
        pl.kernel,
        out_type=jax.ShapeDtypeStruct((NC, N_PAD, D), jnp.float32),
        mesh=mesh,
        scratch_types=[
            pltpu.VMEM((IG, CH), jnp.int32),
            pltpu.VMEM((IG, CH), jnp.int32),
            pltpu.VMEM((CH, D), jnp.float32),
            pltpu.VMEM((CH, D), jnp.float32),
            pltpu.VMEM((CH, D), jnp.float32),
            pltpu.VMEM((CH, D), jnp.float32),
            pltpu.VMEM_SHARED((N_PAD, D), jnp.float32),
        ] + [pltpu.SemaphoreType.DMA] * 6,
    )
    def k(table_hbm, src_hbm, dst_hbm, e_hbm, out_hbm,
          si, di, ev0, ev1, xg0, xg1, agg_sh,
          sg0, sg1, se0, se1, ss0, ss1):
        c = lax.axis_index("c")
        s = lax.axis_index("s")
        ev = (ev0, ev1)
        xg = (xg0, xg1)
        sg = (sg0, sg1)
        se = (se0, se1)
        ss = (ss0, ss1)

        def _zrow(i, carry):
            for cc in range(D // 16):
                ev0[i, pl.ds(cc * 16, 16)] = jnp.zeros((16,), jnp.float32)
            return carry
        lax.fori_loop(0, CH, _zrow, 0)
        zbase = s * ZROWS
        for t in range(ZROWS // CH):
            pltpu.sync_copy(ev0, agg_sh.at[pl.ds(zbase + t * CH, CH)])
        pltpu.sync_copy(ev0.at[pl.ds(0, ZROWS % CH)],
                        agg_sh.at[pl.ds(zbase + (ZROWS // CH) * CH, ZROWS % CH)])
        plsc.subcore_barrier()

        is0 = c == 0
        ng = jnp.where(is0, NG0, NG1)
        base_row = jnp.where(is0, s * NCH0, R0 + s * NCH1)

        def _compute(b):
            def _row(r, rc):
                for cc in range(D // 16):
                    sl = pl.ds(cc * 16, 16)
                    ev[b][r, sl] = jnp.maximum(ev[b][r, sl] + xg[b][r, sl], 0.0)
                return rc
            lax.fori_loop(0, CH, _row, 0, unroll=4)

        def _group(g, carry):
            r0 = base_row + g * IG
            pltpu.sync_copy(src_hbm.at[pl.ds(r0, IG)], si)
            pltpu.sync_copy(dst_hbm.at[pl.ds(r0, IG)], di)
            dg = [None, None]
            de = [None, None]
            dsc = [None, None]
            dg[0] = pltpu.async_copy(table_hbm.at[si.at[0]], xg[0], sg[0])
            de[0] = pltpu.async_copy(e_hbm.at[pl.ds(r0 * CH, CH)], ev[0], se[0])
            for bb in range(IG):
                nb = bb % 2
                ob = 1 - nb
                if bb + 1 < IG:
                    if dsc[ob] is not None:
                        dsc[ob].wait()
                    dg[ob] = pltpu.async_copy(table_hbm.at[si.at[bb + 1]],
                                              xg[ob], sg[ob])
                    de[ob] = pltpu.async_copy(
                        e_hbm.at[pl.ds((r0 + bb + 1) * CH, CH)], ev[ob], se[ob])
                de[nb].wait()
                dg[nb].wait()
                _compute(nb)
                dsc[nb] = pltpu.make_async_copy(ev[nb], agg_sh.at[di.at[bb]],
                                                ss[nb])
                dsc[nb].start(add=True)
            dsc[0].wait()
            dsc[1].wait()
            return carry
        lax.fori_loop(0, ng, _group, 0)

        plsc.subcore_barrier()
        pltpu.sync_copy(agg_sh.at[pl.ds(s * ZROWS, ZROWS)],
                        out_hbm.at[c].at[pl.ds(s * ZROWS, ZROWS)])

    return k(table, src2d, dst2d, e)


def _node_mlp_body(x_ref, a0_ref, a1_ref, wa_ref, ba_ref, wb_ref, bb_ref, o_ref):
    h = x_ref[...] + a0_ref[0] + a1_ref[0]
    h = jnp.dot(h, wa_ref[...], preferred_element_type=jnp.float32) + ba_ref[...]
    h = jnp.dot(h, wb_ref[...], preferred_element_type=jnp.float32) + bb_ref[...]
    o_ref[...] = jnp.maximum(h, 0.0)


def _node_mlp(x, agg, wat, ba, wbt, bb):
    grid = (N // RN,)
    return pl.pallas_call(
        _node_mlp_body,
        grid=grid,
        in_specs=[
            pl.BlockSpec((RN, D), lambda i: (i, 0)),
            pl.BlockSpec((1, RN, D), lambda i: (0, i, 0)),
            pl.BlockSpec((1, RN, D), lambda i: (1, i, 0)),
            pl.BlockSpec((D, D), lambda i: (0, 0)),
            pl.BlockSpec((1, D), lambda i: (0, 0)),
            pl.BlockSpec((D, D), lambda i: (0, 0)),
            pl.BlockSpec((1, D), lambda i: (0, 0)),
        ],
        out_specs=pl.BlockSpec((RN, D), lambda i: (i, 0)),
        out_shape=jax.ShapeDtypeStruct((N, D), jnp.float32),
    )(x, agg, agg, wat, ba.reshape(1, D), wbt, bb.reshape(1, D))


def _node_mlp_pool_body(x_ref, a0_ref, a1_ref, wa_ref, ba_ref, wb_ref, bb_ref,
                        batch_ref, sums_ref, cnts_ref):
    i = pl.program_id(0)
    h = x_ref[...] + a0_ref[0] + a1_ref[0]
    h = jnp.dot(h, wa_ref[...], preferred_element_type=jnp.float32) + ba_ref[...]
    h = jnp.dot(h, wb_ref[...], preferred_element_type=jnp.float32) + bb_ref[...]
    h = jnp.maximum(h, 0.0)
    b = batch_ref[0, 0, :]
    onehot = (b[:, None] == lax.broadcasted_iota(jnp.int32, (RN, G), 1)).astype(jnp.float32)
    part = lax.dot_general(onehot, h, (((0,), (0,)), ((), ())),
                           preferred_element_type=jnp.float32)
    cnt = jnp.broadcast_to(jnp.sum(onehot, axis=0)[:, None], (G, D))

    @pl.when(i == 0)
    def _():
        sums_ref[...] = jnp.zeros_like(sums_ref)
        cnts_ref[...] = jnp.zeros_like(cnts_ref)
    sums_ref[...] += part
    cnts_ref[...] += cnt


def _node_mlp_pool(h1, agg, wat, ba, wbt, bb, batch3d):
    grid = (N // RN,)
    return pl.pallas_call(
        _node_mlp_pool_body,
        grid=grid,
        in_specs=[
            pl.BlockSpec((RN, D), lambda i: (i, 0)),
            pl.BlockSpec((1, RN, D), lambda i: (0, i, 0)),
            pl.BlockSpec((1, RN, D), lambda i: (1, i, 0)),
            pl.BlockSpec((D, D), lambda i: (0, 0)),
            pl.BlockSpec((1, D), lambda i: (0, 0)),
            pl.BlockSpec((D, D), lambda i: (0, 0)),
            pl.BlockSpec((1, D), lambda i: (0, 0)),
            pl.BlockSpec((1, 1, RN), lambda i: (i, 0, 0)),
        ],
        out_specs=[
            pl.BlockSpec((G, D), lambda i: (0, 0)),
            pl.BlockSpec((G, D), lambda i: (0, 0)),
        ],
        out_shape=[
            jax.ShapeDtypeStruct((G, D), jnp.float32),
            jax.ShapeDtypeStruct((G, D), jnp.float32),
        ],
    )(h1, agg, agg, wat, ba.reshape(1, D), wbt, bb.reshape(1, D), batch3d)


def _head_body(sums_ref, cnts_ref, w_ref, b_ref, o_ref):
    pooled = sums_ref[...] / jnp.maximum(cnts_ref[...], 1.0)
    logits = lax.dot_general(pooled, w_ref[...], (((1,), (1,)), ((), ())),
                             preferred_element_type=jnp.float32) + b_ref[...]
    mask = lax.broadcasted_iota(jnp.int32, (G, G), 1) < C
    logits = jnp.where(mask, logits, -1e30)
    m = jnp.max(logits, axis=1, keepdims=True)
    ez = jnp.exp(logits - m)
    o_ref[...] = ez / jnp.sum(ez, axis=1, keepdims=True)


def _head(sums, cnts, ffn_Wp, ffn_bp):
    return pl.pallas_call(
        _head_body,
        in_specs=[
            pl.BlockSpec((G, D), lambda: (0, 0)),
            pl.BlockSpec((G, D), lambda: (0, 0)),
            pl.BlockSpec((G, D), lambda: (0, 0)),
            pl.BlockSpec((1, G), lambda: (0, 0)),
        ],
        out_specs=pl.BlockSpec((G, G), lambda: (0, 0)),
        out_shape=jax.ShapeDtypeStruct((G, G), jnp.float32),
    )(sums, cnts, ffn_Wp, ffn_bp)


def kernel(x, edge_index, edge_attr, batch,
           le1_W, le1_b, n1a_W, n1a_b, n1b_W, n1b_b,
           le2_W, le2_b, n2a_W, n2a_b, n2b_W, n2b_b,
           ffn_W, ffn_b):
    pad = E_PAD - E
    src2d = jnp.concatenate([edge_index[0], jnp.zeros((pad,), jnp.int32)]).reshape(CR_TOT, CH)
    dst2d = jnp.concatenate([edge_index[1], jnp.full((pad,), N, jnp.int32)]).reshape(CR_TOT, CH)
    batch3d = batch.reshape(N // RN, 1, RN)

    e1 = _edge_embed(edge_attr, le1_W.T, le1_b)
    agg1 = _sc_gather_scatter(x, src2d, dst2d, e1)
    e2 = _edge_embed(edge_attr, le2_W.T, le2_b)
    h1 = _node_mlp(x, agg1, n1a_W.T, n1a_b, n1b_W.T, n1b_b)

    agg2 = _sc_gather_scatter(h1, src2d, dst2d, e2)
    ffn_Wp = jnp.concatenate([ffn_W, jnp.zeros((G - C, D), jnp.float32)])
    ffn_bp = jnp.concatenate([ffn_b, jnp.zeros((G - C,), jnp.float32)]).reshape(1, G)
    sums, cnts = _node_mlp_pool(h1, agg2, n2a_W.T, n2a_b, n2b_W.T, n2b_b, batch3d)

    out = _head(sums, cnts, ffn_Wp, ffn_bp)
    return out[:, :C]


# speedup vs baseline: 1.2602x; 1.0823x over previous
import functools

import jax
import jax.numpy as jnp
from jax import lax
from jax.experimental import pallas as pl
from jax.experimental.pallas import tpu as pltpu
from jax.experimental.pallas import tpu_sc as plsc

N = 10000
E = 320000
D = 128
ED = 16
G = 16
C = 10

NC = 2
NS = 16
CH = 80
CR_TOT = E // CH
NCH0 = 160
NCH1 = 96
R0 = NS * NCH0
IG = 16
NG0 = NCH0 // IG
NG1 = NCH1 // IG
N_PAD = 10112
ZROWS = N_PAD // NS

RA = 8000
RN = 1000


def _edge_embed_body(a_ref, w_ref, b_ref, e_ref):
    e_ref[...] = (jnp.dot(a_ref[...], w_ref[...], preferred_element_type=jnp.float32)
                  + b_ref[...])


def _edge_embed(attr, wt, b):
    return pl.pallas_call(
        _edge_embed_body,
        grid=(E // RA,),
        in_specs=[
            pl.BlockSpec((RA, ED), lambda i: (i, 0)),
            pl.BlockSpec((ED, D), lambda i: (0, 0)),
            pl.BlockSpec((1, D), lambda i: (0, 0)),
        ],
        out_specs=pl.BlockSpec((RA, D), lambda i: (i, 0)),
        out_shape=jax.ShapeDtypeStruct((E, D), jnp.float32),
    )(attr, wt, b.reshape(1, D))


def _sc_gather_scatter(table, src2d, dst2d, e):
    mesh = plsc.VectorSubcoreMesh(core_axis_name="c", subcore_axis_name="s")

    @functools.partial(
        pl.kernel,
        out_type=jax.ShapeDtypeStruct((NC, N_PAD, D), jnp.float32),
        mesh=mesh,
        scratch_types=[
            pltpu.VMEM((IG, CH), jnp.int32),
            pltpu.VMEM((IG, CH), jnp.int32),
            pltpu.VMEM((CH, D), jnp.float32),
            pltpu.VMEM((CH, D), jnp.float32),
            pltpu.VMEM((CH, D), jnp.float32),
            pltpu.VMEM((CH, D), jnp.float32),
            pltpu.VMEM_SHARED((N_PAD, D), jnp.float32),
        ] + [pltpu.SemaphoreType.DMA] * 6,
    )
    def k(table_hbm, src_hbm, dst_hbm, e_hbm, out_hbm,
          si, di, ev0, ev1, xg0, xg1, agg_sh,
          sg0, sg1, se0, se1, ss0, ss1):
        c = lax.axis_index("c")
        s = lax.axis_index("s")
        ev = (ev0, ev1)
        xg = (xg0, xg1)
        sg = (sg0, sg1)
        se = (se0, se1)
        ss = (ss0, ss1)

        def _zrow(i, carry):
            for cc in range(D // 16):
                ev0[i, pl.ds(cc * 16, 16)] = jnp.zeros((16,), jnp.float32)
            return carry
        lax.fori_loop(0, CH, _zrow, 0)
        zbase = s * ZROWS
        for t in range(ZROWS // CH):
            pltpu.sync_copy(ev0, agg_sh.at[pl.ds(zbase + t * CH, CH)])
        pltpu.sync_copy(ev0.at[pl.ds(0, ZROWS % CH)],
                        agg_sh.at[pl.ds(zbase + (ZROWS // CH) * CH, ZROWS % CH)])
        plsc.subcore_barrier()

        is0 = c == 0
        ng = jnp.where(is0, NG0, jnp.where(s == NS - 1, 0, NG1))
        base_row = jnp.where(is0, s * NCH0, R0 + s * NCH1)

        def _compute(b):
            def _row(r, rc):
                for cc in range(D // 16):
                    sl = pl.ds(cc * 16, 16)
                    ev[b][r, sl] = jnp.maximum(ev[b][r, sl] + xg[b][r, sl], 0.0)
                return rc
            lax.fori_loop(0, CH, _row, 0, unroll=4)

        def _group(g, carry):
            r0 = base_row + g * IG
            pltpu.sync_copy(src_hbm.at[pl.ds(r0, IG)], si)
            pltpu.sync_copy(dst_hbm.at[pl.ds(r0, IG)], di)
            dg = [None, None]
            de = [None, None]
            dsc = [None, None]
            dg[0] = pltpu.async_copy(table_hbm.at[si.at[0]], xg[0], sg[0])
            de[0] = pltpu.async_copy(e_hbm.at[pl.ds(r0 * CH, CH)], ev[0], se[0])
            for bb in range(IG):
                nb = bb % 2
                ob = 1 - nb
                if bb + 1 < IG:
                    if dsc[ob] is not None:
                        dsc[ob].wait()
                    dg[ob] = pltpu.async_copy(table_hbm.at[si.at[bb + 1]],
                                              xg[ob], sg[ob])
                    de[ob] = pltpu.async_copy(
                        e_hbm.at[pl.ds((r0 + bb + 1) * CH, CH)], ev[ob], se[ob])
                de[nb].wait()
                dg[nb].wait()
                _compute(nb)
                dsc[nb] = pltpu.make_async_copy(ev[nb], agg_sh.at[di.at[bb]],
                                                ss[nb])
                dsc[nb].start(add=True)
            dsc[0].wait()
            dsc[1].wait()
            return carry
        lax.fori_loop(0, ng, _group, 0)

        plsc.subcore_barrier()
        pltpu.sync_copy(agg_sh.at[pl.ds(s * ZROWS, ZROWS)],
                        out_hbm.at[c].at[pl.ds(s * ZROWS, ZROWS)])

    return k(table, src2d, dst2d, e)


def _node_mlp_body(x_ref, a0_ref, a1_ref, wa_ref, ba_ref, wb_ref, bb_ref, o_ref):
    h = x_ref[...] + a0_ref[0] + a1_ref[0]
    h = jnp.dot(h, wa_ref[...], preferred_element_type=jnp.float32) + ba_ref[...]
    h = jnp.dot(h, wb_ref[...], preferred_element_type=jnp.float32) + bb_ref[...]
    o_ref[...] = jnp.maximum(h, 0.0)


def _node_mlp(x, agg, wat, ba, wbt, bb):
    grid = (N // RN,)
    return pl.pallas_call(
        _node_mlp_body,
        grid=grid,
        in_specs=[
            pl.BlockSpec((RN, D), lambda i: (i, 0)),
            pl.BlockSpec((1, RN, D), lambda i: (0, i, 0)),
            pl.BlockSpec((1, RN, D), lambda i: (1, i, 0)),
            pl.BlockSpec((D, D), lambda i: (0, 0)),
            pl.BlockSpec((1, D), lambda i: (0, 0)),
            pl.BlockSpec((D, D), lambda i: (0, 0)),
            pl.BlockSpec((1, D), lambda i: (0, 0)),
        ],
        out_specs=pl.BlockSpec((RN, D), lambda i: (i, 0)),
        out_shape=jax.ShapeDtypeStruct((N, D), jnp.float32),
    )(x, agg, agg, wat, ba.reshape(1, D), wbt, bb.reshape(1, D))


def _node_mlp_pool_body(x_ref, a0_ref, a1_ref, wa_ref, ba_ref, wb_ref, bb_ref,
                        batch_ref, sums_ref, cnts_ref):
    i = pl.program_id(0)
    h = x_ref[...] + a0_ref[0] + a1_ref[0]
    h = jnp.dot(h, wa_ref[...], preferred_element_type=jnp.float32) + ba_ref[...]
    h = jnp.dot(h, wb_ref[...], preferred_element_type=jnp.float32) + bb_ref[...]
    h = jnp.maximum(h, 0.0)
    b = batch_ref[0, 0, :]
    onehot = (b[:, None] == lax.broadcasted_iota(jnp.int32, (RN, G), 1)).astype(jnp.float32)
    part = lax.dot_general(onehot, h, (((0,), (0,)), ((), ())),
                           preferred_element_type=jnp.float32)
    cnt = jnp.broadcast_to(jnp.sum(onehot, axis=0)[:, None], (G, D))

    @pl.when(i == 0)
    def _():
        sums_ref[...] = jnp.zeros_like(sums_ref)
        cnts_ref[...] = jnp.zeros_like(cnts_ref)
    sums_ref[...] += part
    cnts_ref[...] += cnt


def _node_mlp_pool(h1, agg, wat, ba, wbt, bb, batch3d):
    grid = (N // RN,)
    return pl.pallas_call(
        _node_mlp_pool_body,
        grid=grid,
        in_specs=[
            pl.BlockSpec((RN, D), lambda i: (i, 0)),
            pl.BlockSpec((1, RN, D), lambda i: (0, i, 0)),
            pl.BlockSpec((1, RN, D), lambda i: (1, i, 0)),
            pl.BlockSpec((D, D), lambda i: (0, 0)),
            pl.BlockSpec((1, D), lambda i: (0, 0)),
            pl.BlockSpec((D, D), lambda i: (0, 0)),
            pl.BlockSpec((1, D), lambda i: (0, 0)),
            pl.BlockSpec((1, 1, RN), lambda i: (i, 0, 0)),
        ],
        out_specs=[
            pl.BlockSpec((G, D), lambda i: (0, 0)),
            pl.BlockSpec((G, D), lambda i: (0, 0)),
        ],
        out_shape=[
            jax.ShapeDtypeStruct((G, D), jnp.float32),
            jax.ShapeDtypeStruct((G, D), jnp.float32),
        ],
    )(h1, agg, agg, wat, ba.reshape(1, D), wbt, bb.reshape(1, D), batch3d)


def _head_body(sums_ref, cnts_ref, w_ref, b_ref, o_ref):
    pooled = sums_ref[...] / jnp.maximum(cnts_ref[...], 1.0)
    logits = lax.dot_general(pooled, w_ref[...], (((1,), (1,)), ((), ())),
                             preferred_element_type=jnp.float32) + b_ref[...]
    mask = lax.broadcasted_iota(jnp.int32, (G, G), 1) < C
    logits = jnp.where(mask, logits, -1e30)
    m = jnp.max(logits, axis=1, keepdims=True)
    ez = jnp.exp(logits - m)
    o_ref[...] = ez / jnp.sum(ez, axis=1, keepdims=True)


def _head(sums, cnts, ffn_Wp, ffn_bp):
    return pl.pallas_call(
        _head_body,
        in_specs=[
            pl.BlockSpec((G, D), lambda: (0, 0)),
            pl.BlockSpec((G, D), lambda: (0, 0)),
            pl.BlockSpec((G, D), lambda: (0, 0)),
            pl.BlockSpec((1, G), lambda: (0, 0)),
        ],
        out_specs=pl.BlockSpec((G, G), lambda: (0, 0)),
        out_shape=jax.ShapeDtypeStruct((G, G), jnp.float32),
    )(sums, cnts, ffn_Wp, ffn_bp)


def kernel(x, edge_index, edge_attr, batch,
           le1_W, le1_b, n1a_W, n1a_b, n1b_W, n1b_b,
           le2_W, le2_b, n2a_W, n2a_b, n2b_W, n2b_b,
           ffn_W, ffn_b):
    src2d = edge_index[0].reshape(CR_TOT, CH)
    dst2d = edge_index[1].reshape(CR_TOT, CH)
    batch3d = batch.reshape(N // RN, 1, RN)

    e1 = _edge_embed(edge_attr, le1_W.T, le1_b)
    agg1 = _sc_gather_scatter(x, src2d, dst2d, e1)
    e2 = _edge_embed(edge_attr, le2_W.T, le2_b)
    h1 = _node_mlp(x, agg1, n1a_W.T, n1a_b, n1b_W.T, n1b_b)

    agg2 = _sc_gather_scatter(h1, src2d, dst2d, e2)
    ffn_Wp = jnp.concatenate([ffn_W, jnp.zeros((G - C, D), jnp.float32)])
    ffn_bp = jnp.concatenate([ffn_b, jnp.zeros((G - C,), jnp.float32)]).reshape(1, G)
    sums, cnts = _node_mlp_pool(h1, agg2, n2a_W.T, n2a_b, n2b_W.T, n2b_b, batch3d)

    out = _head(sums, cnts, ffn_Wp, ffn_bp)
    return out[:, :C]


# speedup vs baseline: 1.3829x; 1.0974x over previous
import functools

import jax
import jax.numpy as jnp
from jax import lax
from jax.experimental import pallas as pl
from jax.experimental.pallas import tpu as pltpu
from jax.experimental.pallas import tpu_sc as plsc

N = 10000
E = 320000
D = 128
ED = 16
G = 16
C = 10

NC = 2
NS = 16
CH = 80
CR_TOT = E // CH
IG = 16
N_PAD = 10112
ZROWS = N_PAD // NS

RA = 8000
RN = 1000


def _edge_embed_body(a_ref, w_ref, b_ref, e_ref):
    e_ref[...] = (jnp.dot(a_ref[...], w_ref[...], preferred_element_type=jnp.float32)
                  + b_ref[...])


def _edge_embed(attr, wt, b):
    return pl.pallas_call(
        _edge_embed_body,
        grid=(E // RA,),
        in_specs=[
            pl.BlockSpec((RA, ED), lambda i: (i, 0)),
            pl.BlockSpec((ED, D), lambda i: (0, 0)),
            pl.BlockSpec((1, D), lambda i: (0, 0)),
        ],
        out_specs=pl.BlockSpec((RA, D), lambda i: (i, 0)),
        out_shape=jax.ShapeDtypeStruct((E, D), jnp.float32),
    )(attr, wt, b.reshape(1, D))


def _sc_gather_scatter(table, ei3, e):
    mesh = plsc.VectorSubcoreMesh(core_axis_name="c", subcore_axis_name="s")

    @functools.partial(
        pl.kernel,
        out_type=jax.ShapeDtypeStruct((NC, N_PAD, D), jnp.float32),
        mesh=mesh,
        scratch_types=[
            pltpu.VMEM((IG, CH), jnp.int32),
            pltpu.VMEM((IG, CH), jnp.int32),
            pltpu.VMEM((CH, D), jnp.float32),
            pltpu.VMEM((CH, D), jnp.float32),
            pltpu.VMEM((CH, D), jnp.float32),
            pltpu.VMEM((CH, D), jnp.float32),
            pltpu.VMEM_SHARED((N_PAD, D), jnp.float32),
        ] + [pltpu.SemaphoreType.DMA] * 6,
    )
    def k(table_hbm, ei_hbm, e_hbm, out_hbm,
          si, di, ev0, ev1, xg0, xg1, agg_sh,
          sg0, sg1, se0, se1, ss0, ss1):
        c = lax.axis_index("c")
        s = lax.axis_index("s")
        ev = (ev0, ev1)
        xg = (xg0, xg1)
        sg = (sg0, sg1)
        se = (se0, se1)
        ss = (ss0, ss1)

        def _zrow(i, carry):
            for cc in range(D // 16):
                ev0[i, pl.ds(cc * 16, 16)] = jnp.zeros((16,), jnp.float32)
            return carry
        lax.fori_loop(0, CH, _zrow, 0)
        zbase = s * ZROWS
        for t in range(ZROWS // CH):
            pltpu.sync_copy(ev0, agg_sh.at[pl.ds(zbase + t * CH, CH)])
        pltpu.sync_copy(ev0.at[pl.ds(0, ZROWS % CH)],
                        agg_sh.at[pl.ds(zbase + (ZROWS // CH) * CH, ZROWS % CH)])
        plsc.subcore_barrier()

        is0 = c == 0
        ng = jnp.where(is0, jnp.where(s < 2, 9, 8),
                       jnp.where(s == NS - 1, 0, 8))
        base_row = jnp.where(is0,
                             jnp.where(s < 2, 144 * s, 32 + 128 * s),
                             2080 + 128 * s)

        def _compute(b):
            def _row(r, rc):
                for cc in range(D // 16):
                    sl = pl.ds(cc * 16, 16)
                    ev[b][r, sl] = jnp.maximum(ev[b][r, sl] + xg[b][r, sl], 0.0)
                return rc
            lax.fori_loop(0, CH, _row, 0, unroll=4)

        def _group(g, carry):
            r0 = base_row + g * IG
            pltpu.sync_copy(ei_hbm.at[0].at[pl.ds(r0, IG)], si)
            pltpu.sync_copy(ei_hbm.at[1].at[pl.ds(r0, IG)], di)
            dg = [None, None]
            de = [None, None]
            dsc = [None, None]
            dg[0] = pltpu.async_copy(table_hbm.at[si.at[0]], xg[0], sg[0])
            de[0] = pltpu.async_copy(e_hbm.at[pl.ds(r0 * CH, CH)], ev[0], se[0])
            for bb in range(IG):
                nb = bb % 2
                ob = 1 - nb
                if bb + 1 < IG:
                    if dsc[ob] is not None:
                        dsc[ob].wait()
                    dg[ob] = pltpu.async_copy(table_hbm.at[si.at[bb + 1]],
                                              xg[ob], sg[ob])
                    de[ob] = pltpu.async_copy(
                        e_hbm.at[pl.ds((r0 + bb + 1) * CH, CH)], ev[ob], se[ob])
                de[nb].wait()
                dg[nb].wait()
                _compute(nb)
                dsc[nb] = pltpu.make_async_copy(ev[nb], agg_sh.at[di.at[bb]],
                                                ss[nb])
                dsc[nb].start(add=True)
            dsc[0].wait()
            dsc[1].wait()
            return carry
        lax.fori_loop(0, ng, _group, 0)

        plsc.subcore_barrier()
        pltpu.sync_copy(agg_sh.at[pl.ds(s * ZROWS, ZROWS)],
                        out_hbm.at[c].at[pl.ds(s * ZROWS, ZROWS)])

    return k(table, ei3, e)


def _node_mlp_body(x_ref, a0_ref, a1_ref, wa_ref, ba_ref, wb_ref, bb_ref, o_ref):
    h = x_ref[...] + a0_ref[0] + a1_ref[0]
    h = jnp.dot(h, wa_ref[...], preferred_element_type=jnp.float32) + ba_ref[...]
    h = jnp.dot(h, wb_ref[...], preferred_element_type=jnp.float32) + bb_ref[...]
    o_ref[...] = jnp.maximum(h, 0.0)


def _node_mlp(x, agg, wat, ba, wbt, bb):
    grid = (N // RN,)
    return pl.pallas_call(
        _node_mlp_body,
        grid=grid,
        in_specs=[
            pl.BlockSpec((RN, D), lambda i: (i, 0)),
            pl.BlockSpec((1, RN, D), lambda i: (0, i, 0)),
            pl.BlockSpec((1, RN, D), lambda i: (1, i, 0)),
            pl.BlockSpec((D, D), lambda i: (0, 0)),
            pl.BlockSpec((1, D), lambda i: (0, 0)),
            pl.BlockSpec((D, D), lambda i: (0, 0)),
            pl.BlockSpec((1, D), lambda i: (0, 0)),
        ],
        out_specs=pl.BlockSpec((RN, D), lambda i: (i, 0)),
        out_shape=jax.ShapeDtypeStruct((N, D), jnp.float32),
    )(x, agg, agg, wat, ba.reshape(1, D), wbt, bb.reshape(1, D))


def _node_mlp_pool_body(x_ref, a0_ref, a1_ref, wa_ref, ba_ref, wb_ref, bb_ref,
                        batch_ref, sums_ref, cnts_ref):
    i = pl.program_id(0)
    h = x_ref[...] + a0_ref[0] + a1_ref[0]
    h = jnp.dot(h, wa_ref[...], preferred_element_type=jnp.float32) + ba_ref[...]
    h = jnp.dot(h, wb_ref[...], preferred_element_type=jnp.float32) + bb_ref[...]
    h = jnp.maximum(h, 0.0)
    b = batch_ref[0, 0, :]
    onehot = (b[:, None] == lax.broadcasted_iota(jnp.int32, (RN, G), 1)).astype(jnp.float32)
    part = lax.dot_general(onehot, h, (((0,), (0,)), ((), ())),
                           preferred_element_type=jnp.float32)
    cnt = jnp.broadcast_to(jnp.sum(onehot, axis=0)[:, None], (G, D))

    @pl.when(i == 0)
    def _():
        sums_ref[...] = jnp.zeros_like(sums_ref)
        cnts_ref[...] = jnp.zeros_like(cnts_ref)
    sums_ref[...] += part
    cnts_ref[...] += cnt


def _node_mlp_pool(h1, agg, wat, ba, wbt, bb, batch3d):
    grid = (N // RN,)
    return pl.pallas_call(
        _node_mlp_pool_body,
        grid=grid,
        in_specs=[
            pl.BlockSpec((RN, D), lambda i: (i, 0)),
            pl.BlockSpec((1, RN, D), lambda i: (0, i, 0)),
            pl.BlockSpec((1, RN, D), lambda i: (1, i, 0)),
            pl.BlockSpec((D, D), lambda i: (0, 0)),
            pl.BlockSpec((1, D), lambda i: (0, 0)),
            pl.BlockSpec((D, D), lambda i: (0, 0)),
            pl.BlockSpec((1, D), lambda i: (0, 0)),
            pl.BlockSpec((1, 1, RN), lambda i: (i, 0, 0)),
        ],
        out_specs=[
            pl.BlockSpec((G, D), lambda i: (0, 0)),
            pl.BlockSpec((G, D), lambda i: (0, 0)),
        ],
        out_shape=[
            jax.ShapeDtypeStruct((G, D), jnp.float32),
            jax.ShapeDtypeStruct((G, D), jnp.float32),
        ],
    )(h1, agg, agg, wat, ba.reshape(1, D), wbt, bb.reshape(1, D), batch3d)


def _head_body(sums_ref, cnts_ref, w_ref, b_ref, o_ref):
    pooled = sums_ref[...] / jnp.maximum(cnts_ref[...], 1.0)
    logits = lax.dot_general(pooled, w_ref[...], (((1,), (1,)), ((), ())),
                             preferred_element_type=jnp.float32) + b_ref[...]
    mask = lax.broadcasted_iota(jnp.int32, (G, G), 1) < C
    logits = jnp.where(mask, logits, -1e30)
    m = jnp.max(logits, axis=1, keepdims=True)
    ez = jnp.exp(logits - m)
    o_ref[...] = ez / jnp.sum(ez, axis=1, keepdims=True)


def _head(sums, cnts, ffn_Wp, ffn_bp):
    return pl.pallas_call(
        _head_body,
        in_specs=[
            pl.BlockSpec((G, D), lambda: (0, 0)),
            pl.BlockSpec((G, D), lambda: (0, 0)),
            pl.BlockSpec((G, D), lambda: (0, 0)),
            pl.BlockSpec((1, G), lambda: (0, 0)),
        ],
        out_specs=pl.BlockSpec((G, G), lambda: (0, 0)),
        out_shape=jax.ShapeDtypeStruct((G, G), jnp.float32),
    )(sums, cnts, ffn_Wp, ffn_bp)


def kernel(x, edge_index, edge_attr, batch,
           le1_W, le1_b, n1a_W, n1a_b, n1b_W, n1b_b,
           le2_W, le2_b, n2a_W, n2a_b, n2b_W, n2b_b,
           ffn_W, ffn_b):
    ei3 = edge_index.reshape(2, CR_TOT, CH)
    batch3d = batch.reshape(N // RN, 1, RN)

    e1 = _edge_embed(edge_attr, le1_W.T, le1_b)
    agg1 = _sc_gather_scatter(x, ei3, e1)
    e2 = _edge_embed(edge_attr, le2_W.T, le2_b)
    h1 = _node_mlp(x, agg1, n1a_W.T, n1a_b, n1b_W.T, n1b_b)

    agg2 = _sc_gather_scatter(h1, ei3, e2)
    ffn_Wp = jnp.concatenate([ffn_W, jnp.zeros((G - C, D), jnp.float32)])
    ffn_bp = jnp.concatenate([ffn_b, jnp.zeros((G - C,), jnp.float32)]).reshape(1, G)
    sums, cnts = _node_mlp_pool(h1, agg2, n2a_W.T, n2a_b, n2b_W.T, n2b_b, batch3d)

    out = _head(sums, cnts, ffn_Wp, ffn_bp)
    return out[:, :C]


# speedup vs baseline: 1.5066x; 1.0894x over previous
import functools

import jax
import jax.numpy as jnp
from jax import lax
from jax.experimental import pallas as pl
from jax.experimental.pallas import tpu as pltpu
from jax.experimental.pallas import tpu_sc as plsc

N = 10000
E = 320000
D = 128
ED = 16
G = 16
C = 10

NC = 2
NS = 16
CH = 80
CR_TOT = E // CH
IG = 16
N_PAD = 10112
ZROWS = N_PAD // NS

RA = 6400
RN = 1000


def _edge_embed_body(a_ref, w_ref, b_ref, e_ref):
    e_ref[...] = lax.dot_general(a_ref[...], w_ref[...], (((0,), (0,)), ((), ())),
                                 preferred_element_type=jnp.float32) + b_ref[...]


def _edge_embed(attr_t, wt, b):
    return pl.pallas_call(
        _edge_embed_body,
        grid=(E // RA,),
        in_specs=[
            pl.BlockSpec((ED, RA), lambda i: (0, i)),
            pl.BlockSpec((ED, D), lambda i: (0, 0)),
            pl.BlockSpec((1, D), lambda i: (0, 0)),
        ],
        out_specs=pl.BlockSpec((RA, D), lambda i: (i, 0)),
        out_shape=jax.ShapeDtypeStruct((E, D), jnp.float32),
    )(attr_t, wt, b.reshape(1, D))


def _sc_gather_scatter(table, ei3, e):
    mesh = plsc.VectorSubcoreMesh(core_axis_name="c", subcore_axis_name="s")

    @functools.partial(
        pl.kernel,
        out_type=jax.ShapeDtypeStruct((NC, N_PAD, D), jnp.float32),
        mesh=mesh,
        scratch_types=[
            pltpu.VMEM((IG, CH), jnp.int32),
            pltpu.VMEM((IG, CH), jnp.int32),
            pltpu.VMEM((CH, D), jnp.float32),
            pltpu.VMEM((CH, D), jnp.float32),
            pltpu.VMEM((CH, D), jnp.float32),
            pltpu.VMEM((CH, D), jnp.float32),
            pltpu.VMEM_SHARED((N_PAD, D), jnp.float32),
        ] + [pltpu.SemaphoreType.DMA] * 6,
    )
    def k(table_hbm, ei_hbm, e_hbm, out_hbm,
          si, di, ev0, ev1, xg0, xg1, agg_sh,
          sg0, sg1, se0, se1, ss0, ss1):
        c = lax.axis_index("c")
        s = lax.axis_index("s")
        ev = (ev0, ev1)
        xg = (xg0, xg1)
        sg = (sg0, sg1)
        se = (se0, se1)
        ss = (ss0, ss1)

        def _zrow(i, carry):
            for cc in range(D // 16):
                ev0[i, pl.ds(cc * 16, 16)] = jnp.zeros((16,), jnp.float32)
            return carry
        lax.fori_loop(0, CH, _zrow, 0)
        zbase = s * ZROWS
        for t in range(ZROWS // CH):
            pltpu.sync_copy(ev0, agg_sh.at[pl.ds(zbase + t * CH, CH)])
        pltpu.sync_copy(ev0.at[pl.ds(0, ZROWS % CH)],
                        agg_sh.at[pl.ds(zbase + (ZROWS // CH) * CH, ZROWS % CH)])
        plsc.subcore_barrier()

        is0 = c == 0
        ng = jnp.where(is0, jnp.where(s < 2, 9, 8),
                       jnp.where(s == NS - 1, 0, 8))
        base_row = jnp.where(is0,
                             jnp.where(s < 2, 144 * s, 32 + 128 * s),
                             2080 + 128 * s)

        def _compute(b):
            def _row(r, rc):
                for cc in range(D // 16):
                    sl = pl.ds(cc * 16, 16)
                    ev[b][r, sl] = jnp.maximum(ev[b][r, sl] + xg[b][r, sl], 0.0)
                return rc
            lax.fori_loop(0, CH, _row, 0, unroll=4)

        def _group(g, carry):
            r0 = base_row + g * IG
            pltpu.sync_copy(ei_hbm.at[0].at[pl.ds(r0, IG)], si)
            pltpu.sync_copy(ei_hbm.at[1].at[pl.ds(r0, IG)], di)
            dg = [None, None]
            de = [None, None]
            dsc = [None, None]
            dg[0] = pltpu.async_copy(table_hbm.at[si.at[0]], xg[0], sg[0])
            de[0] = pltpu.async_copy(e_hbm.at[pl.ds(r0 * CH, CH)], ev[0], se[0])
            for bb in range(IG):
                nb = bb % 2
                ob = 1 - nb
                if bb + 1 < IG:
                    if dsc[ob] is not None:
                        dsc[ob].wait()
                    dg[ob] = pltpu.async_copy(table_hbm.at[si.at[bb + 1]],
                                              xg[ob], sg[ob])
                    de[ob] = pltpu.async_copy(
                        e_hbm.at[pl.ds((r0 + bb + 1) * CH, CH)], ev[ob], se[ob])
                de[nb].wait()
                dg[nb].wait()
                _compute(nb)
                dsc[nb] = pltpu.make_async_copy(ev[nb], agg_sh.at[di.at[bb]],
                                                ss[nb])
                dsc[nb].start(add=True)
            dsc[0].wait()
            dsc[1].wait()
            return carry
        lax.fori_loop(0, ng, _group, 0)

        plsc.subcore_barrier()
        pltpu.sync_copy(agg_sh.at[pl.ds(s * ZROWS, ZROWS)],
                        out_hbm.at[c].at[pl.ds(s * ZROWS, ZROWS)])

    return k(table, ei3, e)


def _node_mlp_body(x_ref, a0_ref, a1_ref, wa_ref, ba_ref, wb_ref, bb_ref, o_ref):
    h = x_ref[...] + a0_ref[0] + a1_ref[0]
    h = jnp.dot(h, wa_ref[...], preferred_element_type=jnp.float32) + ba_ref[...]
    h = jnp.dot(h, wb_ref[...], preferred_element_type=jnp.float32) + bb_ref[...]
    o_ref[...] = jnp.maximum(h, 0.0)


def _node_mlp(x, agg, wat, ba, wbt, bb):
    grid = (N // RN,)
    return pl.pallas_call(
        _node_mlp_body,
        grid=grid,
        in_specs=[
            pl.BlockSpec((RN, D), lambda i: (i, 0)),
            pl.BlockSpec((1, RN, D), lambda i: (0, i, 0)),
            pl.BlockSpec((1, RN, D), lambda i: (1, i, 0)),
            pl.BlockSpec((D, D), lambda i: (0, 0)),
            pl.BlockSpec((1, D), lambda i: (0, 0)),
            pl.BlockSpec((D, D), lambda i: (0, 0)),
            pl.BlockSpec((1, D), lambda i: (0, 0)),
        ],
        out_specs=pl.BlockSpec((RN, D), lambda i: (i, 0)),
        out_shape=jax.ShapeDtypeStruct((N, D), jnp.float32),
    )(x, agg, agg, wat, ba.reshape(1, D), wbt, bb.reshape(1, D))


def _node_mlp_pool_body(x_ref, a0_ref, a1_ref, wa_ref, ba_ref, wb_ref, bb_ref,
                        batch_ref, sums_ref, cnts_ref):
    i = pl.program_id(0)
    h = x_ref[...] + a0_ref[0] + a1_ref[0]
    h = jnp.dot(h, wa_ref[...], preferred_element_type=jnp.float32) + ba_ref[...]
    h = jnp.dot(h, wb_ref[...], preferred_element_type=jnp.float32) + bb_ref[...]
    h = jnp.maximum(h, 0.0)
    b = batch_ref[0, 0, :]
    onehot = (b[:, None] == lax.broadcasted_iota(jnp.int32, (RN, G), 1)).astype(jnp.float32)
    part = lax.dot_general(onehot, h, (((0,), (0,)), ((), ())),
                           preferred_element_type=jnp.float32)
    cnt = jnp.broadcast_to(jnp.sum(onehot, axis=0)[:, None], (G, D))

    @pl.when(i == 0)
    def _():
        sums_ref[...] = jnp.zeros_like(sums_ref)
        cnts_ref[...] = jnp.zeros_like(cnts_ref)
    sums_ref[...] += part
    cnts_ref[...] += cnt


def _node_mlp_pool(h1, agg, wat, ba, wbt, bb, batch3d):
    grid = (N // RN,)
    return pl.pallas_call(
        _node_mlp_pool_body,
        grid=grid,
        in_specs=[
            pl.BlockSpec((RN, D), lambda i: (i, 0)),
            pl.BlockSpec((1, RN, D), lambda i: (0, i, 0)),
            pl.BlockSpec((1, RN, D), lambda i: (1, i, 0)),
            pl.BlockSpec((D, D), lambda i: (0, 0)),
            pl.BlockSpec((1, D), lambda i: (0, 0)),
            pl.BlockSpec((D, D), lambda i: (0, 0)),
            pl.BlockSpec((1, D), lambda i: (0, 0)),
            pl.BlockSpec((1, 1, RN), lambda i: (i, 0, 0)),
        ],
        out_specs=[
            pl.BlockSpec((G, D), lambda i: (0, 0)),
            pl.BlockSpec((G, D), lambda i: (0, 0)),
        ],
        out_shape=[
            jax.ShapeDtypeStruct((G, D), jnp.float32),
            jax.ShapeDtypeStruct((G, D), jnp.float32),
        ],
    )(h1, agg, agg, wat, ba.reshape(1, D), wbt, bb.reshape(1, D), batch3d)


def _head_body(sums_ref, cnts_ref, w_ref, b_ref, o_ref):
    pooled = sums_ref[...] / jnp.maximum(cnts_ref[...], 1.0)
    logits = lax.dot_general(pooled, w_ref[...], (((1,), (1,)), ((), ())),
                             preferred_element_type=jnp.float32) + b_ref[...]
    mask = lax.broadcasted_iota(jnp.int32, (G, G), 1) < C
    logits = jnp.where(mask, logits, -1e30)
    m = jnp.max(logits, axis=1, keepdims=True)
    ez = jnp.exp(logits - m)
    o_ref[...] = ez / jnp.sum(ez, axis=1, keepdims=True)


def _head(sums, cnts, ffn_Wp, ffn_bp):
    return pl.pallas_call(
        _head_body,
        in_specs=[
            pl.BlockSpec((G, D), lambda: (0, 0)),
            pl.BlockSpec((G, D), lambda: (0, 0)),
            pl.BlockSpec((G, D), lambda: (0, 0)),
            pl.BlockSpec((1, G), lambda: (0, 0)),
        ],
        out_specs=pl.BlockSpec((G, G), lambda: (0, 0)),
        out_shape=jax.ShapeDtypeStruct((G, G), jnp.float32),
    )(sums, cnts, ffn_Wp, ffn_bp)


def kernel(x, edge_index, edge_attr, batch,
           le1_W, le1_b, n1a_W, n1a_b, n1b_W, n1b_b,
           le2_W, le2_b, n2a_W, n2a_b, n2b_W, n2b_b,
           ffn_W, ffn_b):
    ei3 = edge_index.reshape(2, CR_TOT, CH)
    batch3d = batch.reshape(N // RN, 1, RN)

    attr_t = edge_attr.T
    e1 = _edge_embed(attr_t, le1_W.T, le1_b)
    agg1 = _sc_gather_scatter(x, ei3, e1)
    e2 = _edge_embed(attr_t, le2_W.T, le2_b)
    h1 = _node_mlp(x, agg1, n1a_W.T, n1a_b, n1b_W.T, n1b_b)

    agg2 = _sc_gather_scatter(h1, ei3, e2)
    ffn_Wp = jnp.concatenate([ffn_W, jnp.zeros((G - C, D), jnp.float32)])
    ffn_bp = jnp.concatenate([ffn_b, jnp.zeros((G - C,), jnp.float32)]).reshape(1, G)
    sums, cnts = _node_mlp_pool(h1, agg2, n2a_W.T, n2a_b, n2b_W.T, n2b_b, batch3d)

    out = _head(sums, cnts, ffn_Wp, ffn_bp)
    return out[:, :C]


# speedup vs baseline: 1.6658x; 1.1057x over previous
import functools

import jax
import jax.numpy as jnp
from jax import lax
from jax.experimental import pallas as pl
from jax.experimental.pallas import tpu as pltpu
from jax.experimental.pallas import tpu_sc as plsc

N = 10000
E = 320000
D = 128
ED = 16
G = 16
C = 10

NC = 2
NS = 16
CH = 80
CR_TOT = E // CH
IG = 16
N_PAD = 10112
ZROWS = N_PAD // NS

RA = 6400
RN = 1000


def _edge_embed_body(a_ref, w_ref, b_ref, e_ref):
    e_ref[...] = lax.dot_general(a_ref[...], w_ref[...], (((0,), (0,)), ((), ())),
                                 preferred_element_type=jnp.float32) + b_ref[...]


def _edge_embed(attr_t, wt, b):
    return pl.pallas_call(
        _edge_embed_body,
        grid=(E // RA,),
        in_specs=[
            pl.BlockSpec((ED, RA), lambda i: (0, i)),
            pl.BlockSpec((ED, D), lambda i: (0, 0)),
            pl.BlockSpec((1, D), lambda i: (0, 0)),
        ],
        out_specs=pl.BlockSpec((RA, D), lambda i: (i, 0)),
        out_shape=jax.ShapeDtypeStruct((E, D), jnp.float32),
    )(attr_t, wt, b.reshape(1, D))


def _sc_gather_scatter(table, ei3, e):
    mesh = plsc.VectorSubcoreMesh(core_axis_name="c", subcore_axis_name="s")

    @functools.partial(
        pl.kernel,
        out_type=jax.ShapeDtypeStruct((NC, N_PAD, D), jnp.float32),
        mesh=mesh,
        scratch_types=[
            pltpu.VMEM((IG, CH), jnp.int32),
            pltpu.VMEM((IG, CH), jnp.int32),
            pltpu.VMEM((CH, D), jnp.float32),
            pltpu.VMEM((CH, D), jnp.float32),
            pltpu.VMEM((CH, D), jnp.float32),
            pltpu.VMEM((CH, D), jnp.float32),
            pltpu.VMEM_SHARED((N_PAD, D), jnp.float32),
        ] + [pltpu.SemaphoreType.DMA] * 6,
    )
    def k(table_hbm, ei_hbm, e_hbm, out_hbm,
          si, di, ev0, ev1, xg0, xg1, agg_sh,
          sg0, sg1, se0, se1, ss0, ss1):
        c = lax.axis_index("c")
        s = lax.axis_index("s")
        ev = (ev0, ev1)
        xg = (xg0, xg1)
        sg = (sg0, sg1)
        se = (se0, se1)
        ss = (ss0, ss1)

        def _zrow(i, carry):
            for cc in range(D // 16):
                ev0[i, pl.ds(cc * 16, 16)] = jnp.zeros((16,), jnp.float32)
            return carry
        lax.fori_loop(0, CH, _zrow, 0)
        zbase = s * ZROWS
        for t in range(ZROWS // CH):
            pltpu.sync_copy(ev0, agg_sh.at[pl.ds(zbase + t * CH, CH)])
        pltpu.sync_copy(ev0.at[pl.ds(0, ZROWS % CH)],
                        agg_sh.at[pl.ds(zbase + (ZROWS // CH) * CH, ZROWS % CH)])
        plsc.subcore_barrier()

        is0 = c == 0
        ng = jnp.where(is0, 8, jnp.where(s < NS - 2, 8, 5))
        base_row = jnp.where(is0, 128 * s,
                             jnp.where(s == NS - 1, 3920, 2048 + 128 * s))

        def _compute(b):
            def _row(r, rc):
                for cc in range(D // 16):
                    sl = pl.ds(cc * 16, 16)
                    ev[b][r, sl] = jnp.maximum(ev[b][r, sl] + xg[b][r, sl], 0.0)
                return rc
            lax.fori_loop(0, CH, _row, 0, unroll=4)

        def _group(g, carry):
            r0 = base_row + g * IG
            pltpu.sync_copy(ei_hbm.at[0].at[pl.ds(r0, IG)], si)
            pltpu.sync_copy(ei_hbm.at[1].at[pl.ds(r0, IG)], di)
            dg = [None, None]
            de = [None, None]
            dsc = [None, None]
            dg[0] = pltpu.async_copy(table_hbm.at[si.at[0]], xg[0], sg[0])
            de[0] = pltpu.async_copy(e_hbm.at[pl.ds(r0 * CH, CH)], ev[0], se[0])
            for bb in range(IG):
                nb = bb % 2
                ob = 1 - nb
                if bb + 1 < IG:
                    if dsc[ob] is not None:
                        dsc[ob].wait()
                    dg[ob] = pltpu.async_copy(table_hbm.at[si.at[bb + 1]],
                                              xg[ob], sg[ob])
                    de[ob] = pltpu.async_copy(
                        e_hbm.at[pl.ds((r0 + bb + 1) * CH, CH)], ev[ob], se[ob])
                de[nb].wait()
                dg[nb].wait()
                _compute(nb)
                dsc[nb] = pltpu.make_async_copy(ev[nb], agg_sh.at[di.at[bb]],
                                                ss[nb])
                dsc[nb].start(add=True)
            dsc[0].wait()
            dsc[1].wait()
            return carry
        lax.fori_loop(0, ng, _group, 0)

        plsc.subcore_barrier()
        pltpu.sync_copy(agg_sh.at[pl.ds(s * ZROWS, ZROWS)],
                        out_hbm.at[c].at[pl.ds(s * ZROWS, ZROWS)])

    return k(table, ei3, e)


def _node_mlp_body(x_ref, a0_ref, a1_ref, wa_ref, ba_ref, wb_ref, bb_ref, o_ref):
    h = x_ref[...] + a0_ref[0] + a1_ref[0]
    h = jnp.dot(h, wa_ref[...], preferred_element_type=jnp.float32) + ba_ref[...]
    h = jnp.dot(h, wb_ref[...], preferred_element_type=jnp.float32) + bb_ref[...]
    o_ref[...] = jnp.maximum(h, 0.0)


def _node_mlp(x, agg, wat, ba, wbt, bb):
    grid = (N // RN,)
    return pl.pallas_call(
        _node_mlp_body,
        grid=grid,
        in_specs=[
            pl.BlockSpec((RN, D), lambda i: (i, 0)),
            pl.BlockSpec((1, RN, D), lambda i: (0, i, 0)),
            pl.BlockSpec((1, RN, D), lambda i: (1, i, 0)),
            pl.BlockSpec((D, D), lambda i: (0, 0)),
            pl.BlockSpec((1, D), lambda i: (0, 0)),
            pl.BlockSpec((D, D), lambda i: (0, 0)),
            pl.BlockSpec((1, D), lambda i: (0, 0)),
        ],
        out_specs=pl.BlockSpec((RN, D), lambda i: (i, 0)),
        out_shape=jax.ShapeDtypeStruct((N, D), jnp.float32),
    )(x, agg, agg, wat, ba.reshape(1, D), wbt, bb.reshape(1, D))


def _node_mlp_pool_body(x_ref, a0_ref, a1_ref, wa_ref, ba_ref, wb_ref, bb_ref,
                        batch_ref, sums_ref, cnts_ref):
    i = pl.program_id(0)
    h = x_ref[...] + a0_ref[0] + a1_ref[0]
    h = jnp.dot(h, wa_ref[...], preferred_element_type=jnp.float32) + ba_ref[...]
    h = jnp.dot(h, wb_ref[...], preferred_element_type=jnp.float32) + bb_ref[...]
    h = jnp.maximum(h, 0.0)
    b = batch_ref[0, 0, :]
    onehot = (b[:, None] == lax.broadcasted_iota(jnp.int32, (RN, G), 1)).astype(jnp.float32)
    part = lax.dot_general(onehot, h, (((0,), (0,)), ((), ())),
                           preferred_element_type=jnp.float32)
    cnt = jnp.broadcast_to(jnp.sum(onehot, axis=0)[:, None], (G, D))

    @pl.when(i == 0)
    def _():
        sums_ref[...] = jnp.zeros_like(sums_ref)
        cnts_ref[...] = jnp.zeros_like(cnts_ref)
    sums_ref[...] += part
    cnts_ref[...] += cnt


def _node_mlp_pool(h1, agg, wat, ba, wbt, bb, batch3d):
    grid = (N // RN,)
    return pl.pallas_call(
        _node_mlp_pool_body,
        grid=grid,
        in_specs=[
            pl.BlockSpec((RN, D), lambda i: (i, 0)),
            pl.BlockSpec((1, RN, D), lambda i: (0, i, 0)),
            pl.BlockSpec((1, RN, D), lambda i: (1, i, 0)),
            pl.BlockSpec((D, D), lambda i: (0, 0)),
            pl.BlockSpec((1, D), lambda i: (0, 0)),
            pl.BlockSpec((D, D), lambda i: (0, 0)),
            pl.BlockSpec((1, D), lambda i: (0, 0)),
            pl.BlockSpec((1, 1, RN), lambda i: (i, 0, 0)),
        ],
        out_specs=[
            pl.BlockSpec((G, D), lambda i: (0, 0)),
            pl.BlockSpec((G, D), lambda i: (0, 0)),
        ],
        out_shape=[
            jax.ShapeDtypeStruct((G, D), jnp.float32),
            jax.ShapeDtypeStruct((G, D), jnp.float32),
        ],
    )(h1, agg, agg, wat, ba.reshape(1, D), wbt, bb.reshape(1, D), batch3d)


def _head_body(sums_ref, cnts_ref, w_ref, b_ref, o_ref):
    pooled = sums_ref[...] / jnp.maximum(cnts_ref[...], 1.0)
    logits = lax.dot_general(pooled, w_ref[...], (((1,), (1,)), ((), ())),
                             preferred_element_type=jnp.float32) + b_ref[...]
    mask = lax.broadcasted_iota(jnp.int32, (G, G), 1) < C
    logits = jnp.where(mask, logits, -1e30)
    m = jnp.max(logits, axis=1, keepdims=True)
    ez = jnp.exp(logits - m)
    o_ref[...] = ez / jnp.sum(ez, axis=1, keepdims=True)


def _head(sums, cnts, ffn_Wp, ffn_bp):
    return pl.pallas_call(
        _head_body,
        in_specs=[
            pl.BlockSpec((G, D), lambda: (0, 0)),
            pl.BlockSpec((G, D), lambda: (0, 0)),
            pl.BlockSpec((G, D), lambda: (0, 0)),
            pl.BlockSpec((1, G), lambda: (0, 0)),
        ],
        out_specs=pl.BlockSpec((G, G), lambda: (0, 0)),
        out_shape=jax.ShapeDtypeStruct((G, G), jnp.float32),
    )(sums, cnts, ffn_Wp, ffn_bp)


def kernel(x, edge_index, edge_attr, batch,
           le1_W, le1_b, n1a_W, n1a_b, n1b_W, n1b_b,
           le2_W, le2_b, n2a_W, n2a_b, n2b_W, n2b_b,
           ffn_W, ffn_b):
    ei3 = edge_index.reshape(2, CR_TOT, CH)
    batch3d = batch.reshape(N // RN, 1, RN)

    attr_t = edge_attr.T
    e1 = _edge_embed(attr_t, le1_W.T, le1_b)
    agg1 = _sc_gather_scatter(x, ei3, e1)
    e2 = _edge_embed(attr_t, le2_W.T, le2_b)
    h1 = _node_mlp(x, agg1, n1a_W.T, n1a_b, n1b_W.T, n1b_b)

    agg2 = _sc_gather_scatter(h1, ei3, e2)
    ffn_Wp = jnp.concatenate([ffn_W, jnp.zeros((G - C, D), jnp.float32)])
    ffn_bp = jnp.concatenate([ffn_b, jnp.zeros((G - C,), jnp.float32)]).reshape(1, G)
    sums, cnts = _node_mlp_pool(h1, agg2, n2a_W.T, n2a_b, n2b_W.T, n2b_b, batch3d)

    out = _head(sums, cnts, ffn_Wp, ffn_bp)
    return out[:, :C]
